# baseline (speedup 1.0000x reference)
import functools

import jax
import jax.numpy as jnp
from jax import lax
from jax.experimental import pallas as pl
from jax.experimental.pallas import tpu as pltpu
from jax.experimental.pallas import tpu_sc as plsc

N = 10000
E = 320000
D = 128

NC = 2
NS = 16
L = 16
NW = NC * NS
EPW = E // NW
C = 80
NCHUNK = EPW // C
N_PAD = 10240
ROWS_PER_TILE = N_PAD // NS


def _matmul_body(x_ref, w_ref, o_ref):
    o_ref[...] = jnp.dot(x_ref[...], w_ref[...],
                         preferred_element_type=jnp.float32)


def _combine_body(p_ref, b_ref, o_ref):
    o_ref[...] = p_ref[0] + p_ref[1] + b_ref[...]


NBUF = 4
PD = 2


def _sc_body(support_hbm, col_hbm, row_hbm, ew_hbm, out_hbm,
             acc, colv, ewv, rowv, rows, sem_g, sem_c, sem_w, sem_r, sem_s,
             sem_z):
    c = lax.axis_index("c")
    s = lax.axis_index("s")
    wid = s * NC + c
    base = wid * EPW

    def _gather(b):
        return pltpu.make_async_copy(
            support_hbm.at[colv.at[b]], rows.at[b], sem_g.at[b])

    def _colv_dma(i, b):
        return pltpu.make_async_copy(
            col_hbm.at[pl.ds(base + i * C, C)], colv.at[b], sem_c.at[b])

    def _ewv_dma(i, b):
        return pltpu.make_async_copy(
            ew_hbm.at[pl.ds(base + i * C, C)], ewv.at[b], sem_w.at[b])

    def _rowv_dma(i, b):
        return pltpu.make_async_copy(
            row_hbm.at[pl.ds(base + i * C, C)], rowv.at[b], sem_r.at[b])

    def _scatter_start(b):
        pltpu.async_copy(rows.at[b], acc.at[rowv.at[b]], sem_s.at[b],
                         add=True)

    def _scatter_wait(b):
        pltpu.make_async_copy(rows.at[b], acc.at[rowv.at[b]],
                              sem_s.at[b]).wait()

    def _zero_dma(k):
        return pltpu.make_async_copy(
            rows.at[0], acc.at[pl.ds(s * ROWS_PER_TILE + k * C, C)], sem_z)

    def _zero_row(e, _):
        z = jnp.zeros((L,), jnp.float32)
        for j in range(D // L):
            rows[0, e, pl.ds(j * L, L)] = z
        return 0

    lax.fori_loop(0, C, _zero_row, 0)
    for k in range(ROWS_PER_TILE // C):
        _zero_dma(k).start()
    for i in range(2 * PD):
        _colv_dma(i, i).start()
        _ewv_dma(i, i).start()
    for i in range(PD):
        _rowv_dma(i, i).start()
    for k in range(ROWS_PER_TILE // C):
        _zero_dma(k).wait()
    for i in range(PD):
        _colv_dma(i, i).wait()
        _gather(i).start()
    plsc.subcore_barrier()

    def _scale(b):
        @plsc.parallel_loop(0, C // L)
        def _group(g):
            w16 = ewv[b, pl.ds(pl.multiple_of(g * L, L), L)]
            for k in range(L):
                w = jnp.full((L,), w16[k], jnp.float32)
                e = g * L + k
                for j in range(D // L):
                    sl = pl.ds(j * L, L)
                    rows[b, e, sl] = rows[b, e, sl] * w

    def _process(i, b, static_tail=False):
        _gather(b).wait()
        _ewv_dma(i, b).wait()
        _scale(b)
        _rowv_dma(i, b).wait()
        _scatter_start(b)
        bp = (b + PD) % NBUF
        bq = (b + 2 * PD) % NBUF

        def _drain_prev():
            _scatter_wait(bp)

        def _prefetch_near():
            ip = i + PD
            _colv_dma(ip, bp).wait()
            _rowv_dma(ip, bp).start()
            _gather(bp).start()

        def _prefetch_far():
            iq = i + 2 * PD
            _colv_dma(iq, bq).start()
            _ewv_dma(iq, bq).start()

        if static_tail:
            if i >= PD:
                _drain_prev()
            if i + PD < NCHUNK:
                _prefetch_near()
            if i + 2 * PD < NCHUNK:
                _prefetch_far()
        else:
            pl.when(i >= PD)(_drain_prev)
            pl.when(i + PD < NCHUNK)(_prefetch_near)
            pl.when(i + 2 * PD < NCHUNK)(_prefetch_far)

    def _outer(o, _):
        for b in range(NBUF):
            _process(o * NBUF + b, b)
        return 0

    n_main = (NCHUNK // NBUF) * NBUF
    lax.fori_loop(0, NCHUNK // NBUF, _outer, 0)
    for i in range(n_main, NCHUNK):
        _process(i, i % NBUF, static_tail=True)
    for i in range(NCHUNK - PD, NCHUNK):
        _scatter_wait(i % NBUF)
    plsc.subcore_barrier()

    pltpu.sync_copy(acc.at[pl.ds(s * ROWS_PER_TILE, ROWS_PER_TILE)],
                    out_hbm.at[c, pl.ds(s * ROWS_PER_TILE, ROWS_PER_TILE)])


_sc_call = functools.partial(
    pl.kernel,
    out_type=jax.ShapeDtypeStruct((NC, N_PAD, D), jnp.float32),
    mesh=plsc.VectorSubcoreMesh(core_axis_name="c", subcore_axis_name="s"),
    scratch_types=[
        pltpu.VMEM_SHARED((N_PAD, D), jnp.float32),
        pltpu.VMEM((NBUF, C), jnp.int32),
        pltpu.VMEM((NBUF, C), jnp.float32),
        pltpu.VMEM((NBUF, C), jnp.int32),
        pltpu.VMEM((NBUF, C, D), jnp.float32),
        pltpu.SemaphoreType.DMA((NBUF,)),
        pltpu.SemaphoreType.DMA((NBUF,)),
        pltpu.SemaphoreType.DMA((NBUF,)),
        pltpu.SemaphoreType.DMA((NBUF,)),
        pltpu.SemaphoreType.DMA((NBUF,)),
        pltpu.SemaphoreType.DMA,
    ],
)(_sc_body)


def kernel(in_feature, edge_index, edge_weight, weight, bias):
    support = pl.pallas_call(
        _matmul_body,
        grid=(5,),
        in_specs=[
            pl.BlockSpec((N // 5, D), lambda i: (i, 0)),
            pl.BlockSpec((D, D), lambda i: (0, 0)),
        ],
        out_specs=pl.BlockSpec((N // 5, D), lambda i: (i, 0)),
        out_shape=jax.ShapeDtypeStruct((N, D), jnp.float32),
    )(in_feature, weight)

    row = edge_index[0]
    col = edge_index[1]
    partials = _sc_call(support, col, row, edge_weight)

    out = pl.pallas_call(
        _combine_body,
        grid=(5,),
        in_specs=[
            pl.BlockSpec((NC, N // 5, D), lambda i: (0, i, 0)),
            pl.BlockSpec((1, D), lambda i: (0, 0)),
        ],
        out_specs=pl.BlockSpec((N // 5, D), lambda i: (i, 0)),
        out_shape=jax.ShapeDtypeStruct((N, D), jnp.float32),
    )(partials, bias.reshape(1, D))
    return out


# speedup vs baseline: 1.1863x; 1.1863x over previous
import functools

import jax
import jax.numpy as jnp
from jax import lax
from jax.experimental import pallas as pl
from jax.experimental.pallas import tpu as pltpu
from jax.experimental.pallas import tpu_sc as plsc

N = 10000
E = 320000
D = 128

NC = 2
NS = 16
L = 16
NW = NC * NS
EPW = E // NW
C = 80
NCHUNK = EPW // C
N_PAD = 10240
ROWS_PER_TILE = N_PAD // NS


def _matmul_body(x_ref, w_ref, o_ref):
    o_ref[...] = jnp.dot(x_ref[...], w_ref[...],
                         preferred_element_type=jnp.float32)


def _combine_body(p_ref, b_ref, o_ref):
    o_ref[...] = p_ref[0] + p_ref[1] + b_ref[...]


NBUF = 4
PD = 2


def _sc_body(support_hbm, col_hbm, row_hbm, ew_hbm, out_hbm,
             acc, colv, ewv, rowv, rows, sem_g, sem_g2, sem_c, sem_w, sem_r,
             sem_s, sem_z):
    c = lax.axis_index("c")
    s = lax.axis_index("s")
    wid = s * NC + c
    base = wid * EPW

    def _gather(b):
        return pltpu.make_async_copy(
            support_hbm.at[colv.at[b, pl.ds(0, C // 2)]],
            rows.at[b, pl.ds(0, C // 2)], sem_g.at[b])

    def _gather2(b):
        return pltpu.make_async_copy(
            support_hbm.at[colv.at[b, pl.ds(C // 2, C // 2)]],
            rows.at[b, pl.ds(C // 2, C // 2)], sem_g2.at[b])

    def _colv_dma(i, b):
        return pltpu.make_async_copy(
            col_hbm.at[pl.ds(base + i * C, C)], colv.at[b], sem_c.at[b])

    def _ewv_dma(i, b):
        return pltpu.make_async_copy(
            ew_hbm.at[pl.ds(base + i * C, C)], ewv.at[b], sem_w.at[b])

    def _rowv_dma(i, b):
        return pltpu.make_async_copy(
            row_hbm.at[pl.ds(base + i * C, C)], rowv.at[b], sem_r.at[b])

    def _scatter_start(b):
        pltpu.async_copy(rows.at[b], acc.at[rowv.at[b]], sem_s.at[b],
                         add=True)

    def _scatter_wait(b):
        pltpu.make_async_copy(rows.at[b], acc.at[rowv.at[b]],
                              sem_s.at[b]).wait()

    def _zero_dma(k):
        return pltpu.make_async_copy(
            rows.at[0], acc.at[pl.ds(s * ROWS_PER_TILE + k * C, C)], sem_z)

    def _zero_row(e, _):
        z = jnp.zeros((L,), jnp.float32)
        for j in range(D // L):
            rows[0, e, pl.ds(j * L, L)] = z
        return 0

    lax.fori_loop(0, C, _zero_row, 0)
    for k in range(ROWS_PER_TILE // C):
        _zero_dma(k).start()
    for i in range(2 * PD):
        _colv_dma(i, i).start()
        _ewv_dma(i, i).start()
    for i in range(PD):
        _rowv_dma(i, i).start()
    for k in range(ROWS_PER_TILE // C):
        _zero_dma(k).wait()
    for i in range(PD):
        _colv_dma(i, i).wait()
        _gather(i).start()
        _gather2(i).start()
    plsc.subcore_barrier()

    def _scale(b):
        def _group(g, _):
            w16 = ewv[b, pl.ds(pl.multiple_of(g * L, L), L)]
            for k in range(L):
                w = jnp.full((L,), w16[k], jnp.float32)
                e = g * L + k
                for j in range(D // L):
                    sl = pl.ds(j * L, L)
                    rows[b, e, sl] = rows[b, e, sl] * w
            return 0

        lax.fori_loop(0, C // L, _group, 0)

    def _process(i, b, static_tail=False):
        _gather(b).wait()
        _gather2(b).wait()
        _ewv_dma(i, b).wait()
        _scale(b)
        _rowv_dma(i, b).wait()
        _scatter_start(b)
        bp = (b + PD) % NBUF
        bq = (b + 2 * PD) % NBUF

        def _drain_prev():
            _scatter_wait(bp)

        def _prefetch_near():
            ip = i + PD
            _colv_dma(ip, bp).wait()
            _rowv_dma(ip, bp).start()
            _gather(bp).start()
            _gather2(bp).start()

        def _prefetch_far():
            iq = i + 2 * PD
            _colv_dma(iq, bq).start()
            _ewv_dma(iq, bq).start()

        if static_tail:
            if i >= PD:
                _drain_prev()
            if i + PD < NCHUNK:
                _prefetch_near()
            if i + 2 * PD < NCHUNK:
                _prefetch_far()
        else:
            pl.when(i >= PD)(_drain_prev)
            pl.when(i + PD < NCHUNK)(_prefetch_near)
            pl.when(i + 2 * PD < NCHUNK)(_prefetch_far)

    def _outer(o, _):
        for b in range(NBUF):
            _process(o * NBUF + b, b)
        return 0

    n_main = (NCHUNK // NBUF) * NBUF
    lax.fori_loop(0, NCHUNK // NBUF, _outer, 0)
    for i in range(n_main, NCHUNK):
        _process(i, i % NBUF, static_tail=True)
    for i in range(NCHUNK - PD, NCHUNK):
        _scatter_wait(i % NBUF)
    plsc.subcore_barrier()

    pltpu.sync_copy(acc.at[pl.ds(s * ROWS_PER_TILE, ROWS_PER_TILE)],
                    out_hbm.at[c, pl.ds(s * ROWS_PER_TILE, ROWS_PER_TILE)])


_sc_call = functools.partial(
    pl.kernel,
    out_type=jax.ShapeDtypeStruct((NC, N_PAD, D), jnp.float32),
    mesh=plsc.VectorSubcoreMesh(core_axis_name="c", subcore_axis_name="s"),
    scratch_types=[
        pltpu.VMEM_SHARED((N_PAD, D), jnp.float32),
        pltpu.VMEM((NBUF, C), jnp.int32),
        pltpu.VMEM((NBUF, C), jnp.float32),
        pltpu.VMEM((NBUF, C), jnp.int32),
        pltpu.VMEM((NBUF, C, D), jnp.float32),
        pltpu.SemaphoreType.DMA((NBUF,)),
        pltpu.SemaphoreType.DMA((NBUF,)),
        pltpu.SemaphoreType.DMA((NBUF,)),
        pltpu.SemaphoreType.DMA((NBUF,)),
        pltpu.SemaphoreType.DMA((NBUF,)),
        pltpu.SemaphoreType.DMA((NBUF,)),
        pltpu.SemaphoreType.DMA,
    ],
)(_sc_body)


def kernel(in_feature, edge_index, edge_weight, weight, bias):
    support = pl.pallas_call(
        _matmul_body,
        grid=(5,),
        in_specs=[
            pl.BlockSpec((N // 5, D), lambda i: (i, 0)),
            pl.BlockSpec((D, D), lambda i: (0, 0)),
        ],
        out_specs=pl.BlockSpec((N // 5, D), lambda i: (i, 0)),
        out_shape=jax.ShapeDtypeStruct((N, D), jnp.float32),
    )(in_feature, weight)

    row = edge_index[0]
    col = edge_index[1]
    partials = _sc_call(support, col, row, edge_weight)

    out = pl.pallas_call(
        _combine_body,
        grid=(5,),
        in_specs=[
            pl.BlockSpec((NC, N // 5, D), lambda i: (0, i, 0)),
            pl.BlockSpec((1, D), lambda i: (0, 0)),
        ],
        out_specs=pl.BlockSpec((N // 5, D), lambda i: (i, 0)),
        out_shape=jax.ShapeDtypeStruct((N, D), jnp.float32),
    )(partials, bias.reshape(1, D))
    return out


# speedup vs baseline: 1.2169x; 1.0258x over previous
import functools

import jax
import jax.numpy as jnp
from jax import lax
from jax.experimental import pallas as pl
from jax.experimental.pallas import tpu as pltpu
from jax.experimental.pallas import tpu_sc as plsc

N = 10000
E = 320000
D = 128

NC = 2
NS = 16
L = 16
NW = NC * NS
EPW = E // NW
C = 80
NCHUNK = EPW // C
N_PAD = 10240
ROWS_PER_TILE = N_PAD // NS


def _matmul_body(x_ref, w_ref, o_ref):
    o_ref[...] = jnp.dot(x_ref[...], w_ref[...],
                         preferred_element_type=jnp.float32)


def _combine_body(p_ref, b_ref, o_ref):
    o_ref[...] = p_ref[0] + p_ref[1] + b_ref[...]


NBUF = 4
PD = 2


def _sc_body(support_hbm, col_hbm, row_hbm, ew_hbm, out_hbm,
             acc, colv, ewv, rowv, rows, sem_g, sem_g2, sem_c, sem_w, sem_r,
             sem_s, sem_z):
    c = lax.axis_index("c")
    s = lax.axis_index("s")
    wid = s * NC + c
    base = wid * EPW

    def _gather(b):
        return pltpu.make_async_copy(
            support_hbm.at[colv.at[b, pl.ds(0, C // 2)]],
            rows.at[b, pl.ds(0, C // 2)], sem_g.at[b])

    def _gather2(b):
        return pltpu.make_async_copy(
            support_hbm.at[colv.at[b, pl.ds(C // 2, C // 2)]],
            rows.at[b, pl.ds(C // 2, C // 2)], sem_g2.at[b])

    def _colv_dma(i, b):
        return pltpu.make_async_copy(
            col_hbm.at[pl.ds(base + i * C, C)], colv.at[b], sem_c.at[b])

    def _ewv_dma(i, b):
        return pltpu.make_async_copy(
            ew_hbm.at[pl.ds(base + i * C, C)], ewv.at[b], sem_w.at[b])

    def _rowv_dma(i, b):
        return pltpu.make_async_copy(
            row_hbm.at[pl.ds(base + i * C, C)], rowv.at[b], sem_r.at[b])

    def _scatter_start(b):
        pltpu.async_copy(rows.at[b], acc.at[rowv.at[b]], sem_s.at[b],
                         add=True)

    def _scatter_wait(b):
        pltpu.make_async_copy(rows.at[b], acc.at[rowv.at[b]],
                              sem_s.at[b]).wait()

    def _zero_dma(k):
        return pltpu.make_async_copy(
            rows.at[0], acc.at[pl.ds(s * ROWS_PER_TILE + k * C, C)], sem_z)

    def _zero_row(e, _):
        z = jnp.zeros((L,), jnp.float32)
        for j in range(D // L):
            rows[0, e, pl.ds(j * L, L)] = z
        return 0

    lax.fori_loop(0, C, _zero_row, 0)
    for k in range(ROWS_PER_TILE // C):
        _zero_dma(k).start()
    for i in range(2 * PD):
        _colv_dma(i, i).start()
        _ewv_dma(i, i).start()
    for i in range(PD):
        _rowv_dma(i, i).start()
    for k in range(ROWS_PER_TILE // C):
        _zero_dma(k).wait()
    for i in range(PD):
        _colv_dma(i, i).wait()
        _gather(i).start()
        _gather2(i).start()
    plsc.subcore_barrier()

    def _scale(b):
        def _group(g, _):
            w16 = ewv[b, pl.ds(pl.multiple_of(g * L, L), L)]
            for k in range(L):
                w = jnp.full((L,), w16[k], jnp.float32)
                e = g * L + k
                for j in range(D // L):
                    sl = pl.ds(j * L, L)
                    rows[b, e, sl] = rows[b, e, sl] * w
            return 0

        lax.fori_loop(0, C // L, _group, 0)

    def _process(i, b, static_tail=False):
        _gather(b).wait()
        _gather2(b).wait()
        _ewv_dma(i, b).wait()
        _scale(b)
        _rowv_dma(i, b).wait()
        _scatter_start(b)
        bp = (b + PD) % NBUF
        bq = (b + 2 * PD) % NBUF

        def _drain_prev():
            _scatter_wait(bp)

        def _prefetch_near():
            ip = i + PD
            _colv_dma(ip, bp).wait()
            _rowv_dma(ip, bp).start()
            _gather(bp).start()
            _gather2(bp).start()

        def _prefetch_far():
            iq = i + 2 * PD
            _colv_dma(iq, bq).start()
            _ewv_dma(iq, bq).start()

        if static_tail:
            if i >= PD:
                _drain_prev()
            if i + PD < NCHUNK:
                _prefetch_near()
            if i + 2 * PD < NCHUNK:
                _prefetch_far()
        else:
            pl.when(i >= PD)(_drain_prev)
            pl.when(i + PD < NCHUNK)(_prefetch_near)
            pl.when(i + 2 * PD < NCHUNK)(_prefetch_far)

    def _outer(o, _):
        for b in range(NBUF):
            _process(o * NBUF + b, b)
        return 0

    n_main = (NCHUNK // NBUF) * NBUF
    lax.fori_loop(0, NCHUNK // NBUF, _outer, 0)
    for i in range(n_main, NCHUNK):
        _process(i, i % NBUF, static_tail=True)
    for i in range(NCHUNK - PD, NCHUNK):
        _scatter_wait(i % NBUF)
    plsc.subcore_barrier()

    pltpu.sync_copy(acc.at[pl.ds(s * ROWS_PER_TILE, ROWS_PER_TILE)],
                    out_hbm.at[c, pl.ds(s * ROWS_PER_TILE, ROWS_PER_TILE)])


_sc_call = functools.partial(
    pl.kernel,
    out_type=jax.ShapeDtypeStruct((NC, N_PAD, D), jnp.float32),
    mesh=plsc.VectorSubcoreMesh(core_axis_name="c", subcore_axis_name="s"),
    scratch_types=[
        pltpu.VMEM_SHARED((N_PAD, D), jnp.float32),
        pltpu.VMEM((NBUF, C), jnp.int32),
        pltpu.VMEM((NBUF, C), jnp.float32),
        pltpu.VMEM((NBUF, C), jnp.int32),
        pltpu.VMEM((NBUF, C, D), jnp.float32),
        pltpu.SemaphoreType.DMA((NBUF,)),
        pltpu.SemaphoreType.DMA((NBUF,)),
        pltpu.SemaphoreType.DMA((NBUF,)),
        pltpu.SemaphoreType.DMA((NBUF,)),
        pltpu.SemaphoreType.DMA((NBUF,)),
        pltpu.SemaphoreType.DMA((NBUF,)),
        pltpu.SemaphoreType.DMA,
    ],
)(_sc_body)


def kernel(in_feature, edge_index, edge_weight, weight, bias):
    support = pl.pallas_call(
        _matmul_body,
        grid=(2,),
        in_specs=[
            pl.BlockSpec((N // 2, D), lambda i: (i, 0)),
            pl.BlockSpec((D, D), lambda i: (0, 0)),
        ],
        out_specs=pl.BlockSpec((N // 2, D), lambda i: (i, 0)),
        out_shape=jax.ShapeDtypeStruct((N, D), jnp.float32),
    )(in_feature, weight)

    row = edge_index[0]
    col = edge_index[1]
    partials = _sc_call(support, col, row, edge_weight)

    out = pl.pallas_call(
        _combine_body,
        grid=(2,),
        in_specs=[
            pl.BlockSpec((NC, N // 2, D), lambda i: (0, i, 0)),
            pl.BlockSpec((1, D), lambda i: (0, 0)),
        ],
        out_specs=pl.BlockSpec((N // 2, D), lambda i: (i, 0)),
        out_shape=jax.ShapeDtypeStruct((N, D), jnp.float32),
    )(partials, bias.reshape(1, D))
    return out
